# Initial kernel scaffold; baseline (speedup 1.0000x reference)
import jax
import jax.numpy as jnp
from jax.experimental import pallas as pl


def kernel(categorical, numerical, wide_0, wide_1, wide_2, wide_3, wide_4, wide_5, wide_6, wide_7, wide_8, deep_0, deep_1, deep_2, deep_3, deep_4, deep_5, deep_6, deep_7, deep_8, W0, b0, W1, b1, W2, b2):
    raise NotImplementedError("write your pallas kernel here")


# speedup vs baseline: 1.1565x; 1.1565x over previous
import functools

import jax
import jax.numpy as jnp
from jax import lax
from jax.experimental import pallas as pl
from jax.experimental.pallas import tpu as pltpu
from jax.experimental.pallas import tpu_sc as plsc

B = 16384
EMB = 16
NF = 9
NNUM = 8

_info = plsc.get_sparse_core_info()
_NC, _NS = _info.num_cores, _info.num_subcores
_NW = _NC * _NS
_BPW = B // _NW


def _sc_gather_body(cat_ref, *rest):
    wide_refs = rest[0:NF]
    deep_refs = rest[NF:2 * NF]
    deep_out = rest[2 * NF]
    wide_out = rest[2 * NF + 1]
    idx_v, drows_v, wrows_v, dsem, wsem = rest[2 * NF + 2:]

    wid = lax.axis_index("s") * _NC + lax.axis_index("c")
    base = wid * _BPW

    for i in range(NF):
        pltpu.sync_copy(cat_ref.at[pl.ds(i * B + base, _BPW)], idx_v)
        dcp = pltpu.async_copy(deep_refs[i].at[idx_v], drows_v, dsem)
        wcp = pltpu.async_copy(wide_refs[i].at[idx_v], wrows_v, wsem)
        dcp.wait()
        pltpu.sync_copy(drows_v, deep_out.at[pl.ds(i * B + base, _BPW)])
        wcp.wait()
        pltpu.sync_copy(wrows_v, wide_out.at[pl.ds(i * B + base, _BPW)])


@functools.partial(jax.jit, static_argnums=())
def _sc_gather(cat_flat, *tables):
    mesh = plsc.VectorSubcoreMesh(core_axis_name="c", subcore_axis_name="s")
    f = pl.kernel(
        _sc_gather_body,
        out_type=(
            jax.ShapeDtypeStruct((NF * B, EMB), jnp.float32),
            jax.ShapeDtypeStruct((NF * B,), jnp.float32),
        ),
        mesh=mesh,
        scratch_types=[
            pltpu.VMEM((_BPW,), jnp.int32),
            pltpu.VMEM((_BPW, EMB), jnp.float32),
            pltpu.VMEM((_BPW,), jnp.float32),
            pltpu.SemaphoreType.DMA,
            pltpu.SemaphoreType.DMA,
        ],
        compiler_params=pltpu.CompilerParams(use_tc_tiling_on_sc=False),
    )
    return f(cat_flat, *tables)


def _tc_mlp_body(deep_ref, num_ref, wide_ref, w0_ref, b0_ref, w1_ref,
                 b1_ref, w2_ref, b2_ref, out_ref):
    embs = [deep_ref[i] for i in range(NF)]
    x = jnp.concatenate(embs + [num_ref[...]], axis=1)
    h = jnp.maximum(jnp.dot(x, w0_ref[...],
                            preferred_element_type=jnp.float32)
                    + b0_ref[...], 0.0)
    h = jnp.maximum(jnp.dot(h, w1_ref[...],
                            preferred_element_type=jnp.float32)
                    + b1_ref[...], 0.0)
    z = jnp.dot(h, w2_ref[...], preferred_element_type=jnp.float32) \
        + b2_ref[...]
    wide = jnp.sum(wide_ref[...], axis=0)
    logits = z[:, 0] + wide
    out_ref[...] = jax.nn.sigmoid(logits)


def _tc_mlp(deep_g, numerical, wide_g, w0t, b0, w1t, b1, w2t, b2):
    TB = 2048
    grid = (B // TB,)
    return pl.pallas_call(
        _tc_mlp_body,
        grid=grid,
        in_specs=[
            pl.BlockSpec((NF, TB, EMB), lambda t: (0, t, 0)),
            pl.BlockSpec((TB, NNUM), lambda t: (t, 0)),
            pl.BlockSpec((NF, TB), lambda t: (0, t)),
            pl.BlockSpec(w0t.shape, lambda t: (0, 0)),
            pl.BlockSpec(b0.shape, lambda t: (0, 0)),
            pl.BlockSpec(w1t.shape, lambda t: (0, 0)),
            pl.BlockSpec(b1.shape, lambda t: (0, 0)),
            pl.BlockSpec(w2t.shape, lambda t: (0, 0)),
            pl.BlockSpec(b2.shape, lambda t: (0, 0)),
        ],
        out_specs=pl.BlockSpec((TB,), lambda t: (t,)),
        out_shape=jax.ShapeDtypeStruct((B,), jnp.float32),
    )(deep_g, numerical, wide_g, w0t, b0, w1t, b1, w2t, b2)


def kernel(categorical, numerical,
           wide_0, wide_1, wide_2, wide_3, wide_4, wide_5, wide_6, wide_7,
           wide_8,
           deep_0, deep_1, deep_2, deep_3, deep_4, deep_5, deep_6, deep_7,
           deep_8,
           W0, b0, W1, b1, W2, b2):
    wides = (wide_0, wide_1, wide_2, wide_3, wide_4, wide_5, wide_6, wide_7,
             wide_8)
    deeps = (deep_0, deep_1, deep_2, deep_3, deep_4, deep_5, deep_6, deep_7,
             deep_8)
    cat_flat = categorical.T.astype(jnp.int32).reshape(-1)
    wides_flat = tuple(w.reshape(-1) for w in wides)
    deep_g, wide_g = _sc_gather(cat_flat, *wides_flat, *deeps)
    out = _tc_mlp(deep_g.reshape(NF, B, EMB), numerical, wide_g.reshape(NF, B),
                  W0.T, b0.reshape(1, -1), W1.T, b1.reshape(1, -1),
                  W2.T, b2.reshape(1, -1))
    return out
